# baseline (speedup 1.0000x reference)
import functools

import numpy as np
import jax
import jax.numpy as jnp
from jax import lax
from jax.experimental import pallas as pl
from jax.experimental.pallas import tpu as pltpu
from jax.experimental.pallas import tpu_sc as plsc

BS = 4
N = 8192
M = 2048
C = 128
COUT = C + 3
K = 32
MW = 256
NGROUP = MW // 16
G = 4
NV = MW * K // 16
R2 = np.float32(0.2 * 0.2)

SLOT = 8192
NSLOT = 8
CXOFF = NSLOT * SLOT
ARENA_WORDS = NSLOT * SLOT + 3 * MW

_DNUMS = lax.GatherDimensionNumbers(
    offset_dims=(), collapsed_slice_dims=(0,), start_index_map=(0,))


def _bcast(vec, t):
    return lax.gather(vec, jnp.full((16, 1), t, jnp.int32), _DNUMS,
                      slice_sizes=(1,),
                      mode=lax.GatherScatterMode.PROMISE_IN_BOUNDS)


def _sc_query_group(xyzc, childc, feats):
    mesh = plsc.VectorSubcoreMesh(core_axis_name="c", subcore_axis_name="s")

    @functools.partial(
        pl.kernel,
        out_type=(
            jax.ShapeDtypeStruct((BS * COUT, K, M), jnp.float32),
            jax.ShapeDtypeStruct((BS * M * K,), jnp.int32),
        ),
        mesh=mesh,
        compiler_params=pltpu.CompilerParams(needs_layout_passes=False),
        scratch_types=[
            pltpu.VMEM((ARENA_WORDS,), jnp.float32),
            pltpu.VMEM((G, K, MW), jnp.float32),
            pltpu.VMEM((MW * K,), jnp.int32),
            pltpu.VMEM((MW * K,), jnp.int32),
            [pltpu.SemaphoreType.DMA] * G,
            [pltpu.SemaphoreType.DMA] * G,
        ],
    )
    def body(xyzc_ref, childc_ref, feats_ref, grouped_ref, filled_ref,
             arena, ob2, idxb, fillb, sem_t, sem_o):
        wid = lax.axis_index("s") * 2 + lax.axis_index("c")
        b = wid // 8
        mbase = (wid % 8) * MW
        obase = mbase * K

        def slot(s):
            return arena.at[pl.ds(s * SLOT, SLOT)]

        def gslice(ch, r):
            return grouped_ref.at[b * COUT + ch, pl.ds(r * 8, 8),
                                  pl.ds(mbase, MW)]

        def tsrc(ch):
            return feats_ref.at[pl.ds((b * C + ch) * N, N)]

        for d in range(3):
            pltpu.sync_copy(xyzc_ref.at[pl.ds((b * 3 + d) * N, N)], slot(d))
            pltpu.sync_copy(childc_ref.at[pl.ds((b * 3 + d) * M + mbase, MW)],
                            arena.at[pl.ds(CXOFF + d * MW, MW)])

        iota16 = lax.iota(jnp.int32, 16)
        zeros16 = jnp.zeros((16,), jnp.int32)

        def zstep(v, _):
            idxb[pl.ds(v * 16, 16)] = zeros16
            return 0
        lax.fori_loop(0, NV, zstep, 0)

        def group_body(g, _):
            base = g * 16
            cxv = arena[pl.ds(CXOFF + base, 16)]
            cyv = arena[pl.ds(CXOFF + MW + base, 16)]
            czv = arena[pl.ds(CXOFF + 2 * MW + base, 16)]
            rowbase = (base + iota16) * K

            def ocond(carry):
                j0, ptrv, done = carry
                return jnp.logical_and(j0 < N, jnp.logical_not(done))

            def obody(carry):
                j0, ptrv, _ = carry
                for u in range(2):
                    jc = j0 + u * 16
                    xc = arena[pl.ds(jc, 16)]
                    yc = arena[pl.ds(SLOT + jc, 16)]
                    zc = arena[pl.ds(2 * SLOT + jc, 16)]
                    jbase = jnp.full((16,), jc, jnp.int32)
                    for t in range(16):
                        dx = cxv - _bcast(xc, t)
                        dy = cyv - _bcast(yc, t)
                        dz = czv - _bcast(zc, t)
                        d2 = (dx * dx + dy * dy) + dz * dz
                        msk = d2 <= R2
                        okm = jnp.logical_and(msk, ptrv < K)
                        plsc.store_scatter(idxb, [rowbase + ptrv], jbase + t,
                                           mask=okm)
                        ptrv = ptrv + msk.astype(jnp.int32)
                ndone = jnp.sum((ptrv >= K).astype(jnp.int32))
                return (j0 + 32, ptrv, ndone >= 16)

            _, ptrv, _ = lax.while_loop(
                ocond, obody, (jnp.int32(0), zeros16, jnp.bool_(False)))
            cnt = jnp.minimum(ptrv, K)

            def fstep(s, _):
                plsc.store_scatter(fillb, [rowbase + s],
                                   (cnt > s).astype(jnp.int32))
                return 0
            lax.fori_loop(0, K, fstep, 0)
            return 0
        lax.fori_loop(0, NGROUP, group_body, 0)

        pltpu.sync_copy(fillb, filled_ref.at[pl.ds(b * M * K + obase, MW * K)])

        for q in range(G):
            pltpu.async_copy(tsrc(q), slot(4 + q), sem_t[q])

        sv0 = iota16
        sv1 = iota16 + 16

        idx_init = (idxb[pl.ds(0, 16)], idxb[pl.ds(16, 16)],
                    idxb[pl.ds(K, 16)], idxb[pl.ds(K + 16, 16)])

        def next_idx(i):
            i2 = (i + 2) & (MW - 1)
            return (idxb[pl.ds(i2 * K, 16)], idxb[pl.ds(i2 * K + 16, 16)])

        def xstep(i, carry):
            idx0, idx1, idxn0, idxn1 = carry
            iv = jnp.full((16,), i, jnp.int32)
            for q, coff in ((0, CXOFF), (1, CXOFF + MW), (2, CXOFF + 2 * MW)):
                cval = plsc.load_gather(
                    arena, [jnp.full((16,), coff + i, jnp.int32)])
                plsc.store_scatter(ob2.at[q], [sv0, iv],
                                   plsc.load_gather(slot(q), [idx0]) - cval)
                plsc.store_scatter(ob2.at[q], [sv1, iv],
                                   plsc.load_gather(slot(q), [idx1]) - cval)
            return (idxn0, idxn1) + next_idx(i)
        lax.fori_loop(0, MW, xstep, idx_init)
        for q in range(3):
            for r in range(4):
                pltpu.sync_copy(ob2.at[q, pl.ds(r * 8, 8), :], gslice(q, r))

        def dgroup(t2, _):
            for par, tbase in ((0, 4), (1, 0)):
                gi = 2 * t2 + par
                ch0 = gi * G
                for q in range(G):
                    pltpu.make_async_copy(tsrc(ch0 + q), slot(tbase + q),
                                          sem_t[q]).wait()
                nbase = 4 - tbase

                @pl.when(gi + 1 < C // G)
                def _():
                    for q in range(G):
                        pltpu.async_copy(tsrc(ch0 + G + q), slot(nbase + q),
                                         sem_t[q])

                @pl.when(gi >= 1)
                def _():
                    for q in range(G):
                        for r in range(4):
                            pltpu.make_async_copy(
                                ob2.at[q, pl.ds(r * 8, 8), :],
                                gslice(3 + ch0 - G + q, r),
                                sem_o[q]).wait()

                def gbody(i, carry, tbase=tbase):
                    idx0, idx1, idxn0, idxn1 = carry
                    iv = jnp.full((16,), i, jnp.int32)
                    for q in range(G):
                        plsc.store_scatter(
                            ob2.at[q], [sv0, iv],
                            plsc.load_gather(slot(tbase + q), [idx0]))
                        plsc.store_scatter(
                            ob2.at[q], [sv1, iv],
                            plsc.load_gather(slot(tbase + q), [idx1]))
                    return (idxn0, idxn1) + next_idx(i)
                lax.fori_loop(0, MW, gbody, idx_init)

                for q in range(G):
                    for r in range(4):
                        pltpu.async_copy(ob2.at[q, pl.ds(r * 8, 8), :],
                                         gslice(3 + ch0 + q, r), sem_o[q])
            return 0
        lax.fori_loop(0, C // (2 * G), dgroup, 0)

        for q in range(G):
            for r in range(4):
                pltpu.make_async_copy(ob2.at[q, pl.ds(r * 8, 8), :],
                                      gslice(3 + C - G + q, r),
                                      sem_o[q]).wait()

    return body(xyzc, childc, feats)


def kernel(xyz, child_xyz, feats):
    xyzc = jnp.transpose(xyz, (0, 2, 1)).reshape(-1)
    childc = jnp.transpose(child_xyz, (0, 2, 1)).reshape(-1)
    grouped3, filled1 = _sc_query_group(xyzc, childc, feats.reshape(-1))
    grouped = grouped3.reshape(BS, COUT, K, M).transpose(0, 1, 3, 2)
    filled = filled1.reshape(BS, M, K).astype(jnp.bool_)
    return grouped, filled


# speedup vs baseline: 1.0560x; 1.0560x over previous
import functools

import numpy as np
import jax
import jax.numpy as jnp
from jax import lax
from jax.experimental import pallas as pl
from jax.experimental.pallas import tpu as pltpu
from jax.experimental.pallas import tpu_sc as plsc

BS = 4
N = 8192
M = 2048
C = 128
COUT = C + 3
K = 32
MW = 256
NGROUP = MW // 16
G = 4
NV = MW * K // 16
R2 = np.float32(0.2 * 0.2)

SLOT = 8192
NSLOT = 8
CXOFF = NSLOT * SLOT
ARENA_WORDS = NSLOT * SLOT + 3 * MW

_DNUMS = lax.GatherDimensionNumbers(
    offset_dims=(), collapsed_slice_dims=(0,), start_index_map=(0,))


def _bcast(vec, t):
    return lax.gather(vec, jnp.full((16, 1), t, jnp.int32), _DNUMS,
                      slice_sizes=(1,),
                      mode=lax.GatherScatterMode.PROMISE_IN_BOUNDS)


def _sc_query_group(xyzc, childc, feats):
    mesh = plsc.VectorSubcoreMesh(core_axis_name="c", subcore_axis_name="s")

    @functools.partial(
        pl.kernel,
        out_type=(
            jax.ShapeDtypeStruct((BS * COUT, K, M), jnp.float32),
            jax.ShapeDtypeStruct((BS * M * K,), jnp.int32),
        ),
        mesh=mesh,
        compiler_params=pltpu.CompilerParams(needs_layout_passes=False),
        scratch_types=[
            pltpu.VMEM((ARENA_WORDS,), jnp.float32),
            pltpu.VMEM((G, K, MW), jnp.float32),
            pltpu.VMEM((MW * K,), jnp.int32),
            pltpu.VMEM((MW * K,), jnp.int32),
            [pltpu.SemaphoreType.DMA] * G,
            [pltpu.SemaphoreType.DMA] * G,
        ],
    )
    def body(xyzc_ref, childc_ref, feats_ref, grouped_ref, filled_ref,
             arena, ob2, idxb, fillb, sem_t, sem_o):
        wid = lax.axis_index("s") * 2 + lax.axis_index("c")
        b = wid // 8
        mbase = (wid % 8) * MW
        obase = mbase * K

        def slot(s):
            return arena.at[pl.ds(s * SLOT, SLOT)]

        def gslice(ch, r):
            return grouped_ref.at[b * COUT + ch, pl.ds(r * 8, 8),
                                  pl.ds(mbase, MW)]

        def tsrc(ch):
            return feats_ref.at[pl.ds((b * C + ch) * N, N)]

        for d in range(3):
            pltpu.sync_copy(xyzc_ref.at[pl.ds((b * 3 + d) * N, N)], slot(d))
            pltpu.sync_copy(childc_ref.at[pl.ds((b * 3 + d) * M + mbase, MW)],
                            arena.at[pl.ds(CXOFF + d * MW, MW)])

        iota16 = lax.iota(jnp.int32, 16)
        zeros16 = jnp.zeros((16,), jnp.int32)

        def zstep(v, _):
            idxb[pl.ds(v * 16, 16)] = zeros16
            return 0
        lax.fori_loop(0, NV, zstep, 0)

        def group_body(g, _):
            base = g * 16
            cxv = arena[pl.ds(CXOFF + base, 16)]
            cyv = arena[pl.ds(CXOFF + MW + base, 16)]
            czv = arena[pl.ds(CXOFF + 2 * MW + base, 16)]
            rowbase = (base + iota16) * K

            def ocond(carry):
                j0, ptrv, done = carry
                return jnp.logical_and(j0 < N, jnp.logical_not(done))

            def obody(carry):
                j0, ptrv, _ = carry
                for u in range(2):
                    jc = j0 + u * 16
                    xc = arena[pl.ds(jc, 16)]
                    yc = arena[pl.ds(SLOT + jc, 16)]
                    zc = arena[pl.ds(2 * SLOT + jc, 16)]
                    jbase = jnp.full((16,), jc, jnp.int32)
                    for t in range(16):
                        dx = cxv - _bcast(xc, t)
                        dy = cyv - _bcast(yc, t)
                        dz = czv - _bcast(zc, t)
                        d2 = (dx * dx + dy * dy) + dz * dz
                        msk = d2 <= R2
                        okm = jnp.logical_and(msk, ptrv < K)
                        plsc.store_scatter(idxb, [rowbase + ptrv], jbase + t,
                                           mask=okm)
                        ptrv = ptrv + msk.astype(jnp.int32)
                ndone = jnp.sum((ptrv >= K).astype(jnp.int32))
                return (j0 + 32, ptrv, ndone >= 16)

            _, ptrv, _ = lax.while_loop(
                ocond, obody, (jnp.int32(0), zeros16, jnp.bool_(False)))
            cnt = jnp.minimum(ptrv, K)

            def fstep(s, _):
                plsc.store_scatter(fillb, [rowbase + s],
                                   (cnt > s).astype(jnp.int32))
                return 0
            lax.fori_loop(0, K, fstep, 0)
            return 0
        lax.fori_loop(0, NGROUP, group_body, 0)

        pltpu.sync_copy(fillb, filled_ref.at[pl.ds(b * M * K + obase, MW * K)])

        for q in range(G):
            pltpu.async_copy(tsrc(q), slot(4 + q), sem_t[q])

        sv0 = iota16
        sv1 = iota16 + 16

        idx_init = (idxb[pl.ds(0, 16)], idxb[pl.ds(16, 16)],
                    idxb[pl.ds(K, 16)], idxb[pl.ds(K + 16, 16)])

        def next_idx(i):
            i2 = (i + 2) & (MW - 1)
            return (idxb[pl.ds(i2 * K, 16)], idxb[pl.ds(i2 * K + 16, 16)])

        def xstep(i, carry):
            idx0, idx1, idxn0, idxn1 = carry
            iv = jnp.full((16,), i, jnp.int32)
            coffs = (CXOFF, CXOFF + MW, CXOFF + 2 * MW)
            cvals = [plsc.load_gather(
                arena, [jnp.full((16,), coff + i, jnp.int32)])
                for coff in coffs]
            g0 = [plsc.load_gather(slot(q), [idx0]) - cvals[q]
                  for q in range(3)]
            g1 = [plsc.load_gather(slot(q), [idx1]) - cvals[q]
                  for q in range(3)]
            for q in range(3):
                plsc.store_scatter(ob2.at[q], [sv0, iv], g0[q])
                plsc.store_scatter(ob2.at[q], [sv1, iv], g1[q])
            return (idxn0, idxn1) + next_idx(i)
        lax.fori_loop(0, MW, xstep, idx_init)
        for q in range(3):
            for r in range(4):
                pltpu.sync_copy(ob2.at[q, pl.ds(r * 8, 8), :], gslice(q, r))

        def dgroup(t2, _):
            for par, tbase in ((0, 4), (1, 0)):
                gi = 2 * t2 + par
                ch0 = gi * G
                for q in range(G):
                    pltpu.make_async_copy(tsrc(ch0 + q), slot(tbase + q),
                                          sem_t[q]).wait()
                nbase = 4 - tbase

                @pl.when(gi + 1 < C // G)
                def _():
                    for q in range(G):
                        pltpu.async_copy(tsrc(ch0 + G + q), slot(nbase + q),
                                         sem_t[q])

                @pl.when(gi >= 1)
                def _():
                    for q in range(G):
                        for r in range(4):
                            pltpu.make_async_copy(
                                ob2.at[q, pl.ds(r * 8, 8), :],
                                gslice(3 + ch0 - G + q, r),
                                sem_o[q]).wait()

                def gbody(i, carry, tbase=tbase):
                    idx0, idx1, idxn0, idxn1 = carry
                    iv = jnp.full((16,), i, jnp.int32)
                    g0 = [plsc.load_gather(slot(tbase + q), [idx0])
                          for q in range(G)]
                    g1 = [plsc.load_gather(slot(tbase + q), [idx1])
                          for q in range(G)]
                    for q in range(G):
                        plsc.store_scatter(ob2.at[q], [sv0, iv], g0[q])
                        plsc.store_scatter(ob2.at[q], [sv1, iv], g1[q])
                    return (idxn0, idxn1) + next_idx(i)
                lax.fori_loop(0, MW, gbody, idx_init)

                for q in range(G):
                    for r in range(4):
                        pltpu.async_copy(ob2.at[q, pl.ds(r * 8, 8), :],
                                         gslice(3 + ch0 + q, r), sem_o[q])
            return 0
        lax.fori_loop(0, C // (2 * G), dgroup, 0)

        for q in range(G):
            for r in range(4):
                pltpu.make_async_copy(ob2.at[q, pl.ds(r * 8, 8), :],
                                      gslice(3 + C - G + q, r),
                                      sem_o[q]).wait()

    return body(xyzc, childc, feats)


def kernel(xyz, child_xyz, feats):
    xyzc = jnp.transpose(xyz, (0, 2, 1)).reshape(-1)
    childc = jnp.transpose(child_xyz, (0, 2, 1)).reshape(-1)
    grouped3, filled1 = _sc_query_group(xyzc, childc, feats.reshape(-1))
    grouped = grouped3.reshape(BS, COUT, K, M).transpose(0, 1, 3, 2)
    filled = filled1.reshape(BS, M, K).astype(jnp.bool_)
    return grouped, filled


# speedup vs baseline: 1.0618x; 1.0055x over previous
import functools

import numpy as np
import jax
import jax.numpy as jnp
from jax import lax
from jax.experimental import pallas as pl
from jax.experimental.pallas import tpu as pltpu
from jax.experimental.pallas import tpu_sc as plsc

BS = 4
N = 8192
M = 2048
C = 128
COUT = C + 3
K = 32
MW = 256
NGROUP = MW // 16
G = 4
NV = MW * K // 16
R2 = np.float32(0.2 * 0.2)

SLOT = 8192
NSLOT = 8
CXOFF = NSLOT * SLOT
ARENA_WORDS = NSLOT * SLOT + 3 * MW

_DNUMS = lax.GatherDimensionNumbers(
    offset_dims=(), collapsed_slice_dims=(0,), start_index_map=(0,))


def _bcast(vec, t):
    return lax.gather(vec, jnp.full((16, 1), t, jnp.int32), _DNUMS,
                      slice_sizes=(1,),
                      mode=lax.GatherScatterMode.PROMISE_IN_BOUNDS)


def _sc_query_group(xyzc, childc, feats):
    mesh = plsc.VectorSubcoreMesh(core_axis_name="c", subcore_axis_name="s")

    @functools.partial(
        pl.kernel,
        out_type=(
            jax.ShapeDtypeStruct((BS * COUT, K, M), jnp.float32),
            jax.ShapeDtypeStruct((BS * M * K,), jnp.int32),
        ),
        mesh=mesh,
        compiler_params=pltpu.CompilerParams(needs_layout_passes=False),
        scratch_types=[
            pltpu.VMEM((ARENA_WORDS,), jnp.float32),
            pltpu.VMEM((G, K, MW), jnp.float32),
            pltpu.VMEM((MW * K,), jnp.int32),
            pltpu.VMEM((MW * K,), jnp.int32),
            [pltpu.SemaphoreType.DMA] * G,
            [pltpu.SemaphoreType.DMA] * G,
        ],
    )
    def body(xyzc_ref, childc_ref, feats_ref, grouped_ref, filled_ref,
             arena, ob2, idxb, fillb, sem_t, sem_o):
        wid = lax.axis_index("s") * 2 + lax.axis_index("c")
        b = wid // 8
        mbase = (wid % 8) * MW
        obase = mbase * K

        def slot(s):
            return arena.at[pl.ds(s * SLOT, SLOT)]

        def gslice(ch, r):
            return grouped_ref.at[b * COUT + ch, pl.ds(r * 8, 8),
                                  pl.ds(mbase, MW)]

        def tsrc(ch):
            return feats_ref.at[pl.ds((b * C + ch) * N, N)]

        for d in range(3):
            pltpu.sync_copy(xyzc_ref.at[pl.ds((b * 3 + d) * N, N)], slot(d))
            pltpu.sync_copy(childc_ref.at[pl.ds((b * 3 + d) * M + mbase, MW)],
                            arena.at[pl.ds(CXOFF + d * MW, MW)])

        iota16 = lax.iota(jnp.int32, 16)
        zeros16 = jnp.zeros((16,), jnp.int32)

        def zstep(v, _):
            idxb[pl.ds(v * 16, 16)] = zeros16
            return 0
        lax.fori_loop(0, NV, zstep, 0)

        def group_body(g, jmax):
            base = g * 16
            cxv = arena[pl.ds(CXOFF + base, 16)]
            cyv = arena[pl.ds(CXOFF + MW + base, 16)]
            czv = arena[pl.ds(CXOFF + 2 * MW + base, 16)]
            rowbase = (base + iota16) * K

            def ocond(carry):
                j0, ptrv, done = carry
                return jnp.logical_and(j0 < N, jnp.logical_not(done))

            def obody(carry):
                j0, ptrv, _ = carry
                for u in range(2):
                    jc = j0 + u * 16
                    xc = arena[pl.ds(jc, 16)]
                    yc = arena[pl.ds(SLOT + jc, 16)]
                    zc = arena[pl.ds(2 * SLOT + jc, 16)]
                    jbase = jnp.full((16,), jc, jnp.int32)
                    for t in range(16):
                        dx = cxv - _bcast(xc, t)
                        dy = cyv - _bcast(yc, t)
                        dz = czv - _bcast(zc, t)
                        d2 = (dx * dx + dy * dy) + dz * dz
                        msk = d2 <= R2
                        okm = jnp.logical_and(msk, ptrv < K)
                        plsc.store_scatter(idxb, [rowbase + ptrv], jbase + t,
                                           mask=okm)
                        ptrv = ptrv + msk.astype(jnp.int32)
                ndone = jnp.sum((ptrv >= K).astype(jnp.int32))
                return (j0 + 32, ptrv, ndone >= 16)

            jfin, ptrv, _ = lax.while_loop(
                ocond, obody, (jnp.int32(0), zeros16, jnp.bool_(False)))
            cnt = jnp.minimum(ptrv, K)

            def fstep(s, _):
                plsc.store_scatter(fillb, [rowbase + s],
                                   (cnt > s).astype(jnp.int32))
                return 0
            lax.fori_loop(0, K, fstep, 0)
            return jnp.maximum(jmax, jfin)
        jmax = lax.fori_loop(0, NGROUP, group_body, jnp.int32(0))

        pltpu.sync_copy(fillb, filled_ref.at[pl.ds(b * M * K + obase, MW * K)])

        TCH = 2048

        def tload(ch, tref, sem):
            src = tsrc(ch)
            pltpu.async_copy(src.at[pl.ds(0, TCH)], tref.at[pl.ds(0, TCH)], sem)
            for c in range(1, N // TCH):
                @pl.when(jmax > c * TCH)
                def _(c=c):
                    pltpu.async_copy(src.at[pl.ds(c * TCH, TCH)],
                                     tref.at[pl.ds(c * TCH, TCH)], sem)

        def twait(ch, tref, sem):
            src = tsrc(ch)
            pltpu.make_async_copy(src.at[pl.ds(0, TCH)],
                                  tref.at[pl.ds(0, TCH)], sem).wait()
            for c in range(1, N // TCH):
                @pl.when(jmax > c * TCH)
                def _(c=c):
                    pltpu.make_async_copy(src.at[pl.ds(c * TCH, TCH)],
                                          tref.at[pl.ds(c * TCH, TCH)],
                                          sem).wait()

        for q in range(G):
            tload(q, slot(4 + q), sem_t[q])

        sv0 = iota16
        sv1 = iota16 + 16

        idx_init = (idxb[pl.ds(0, 16)], idxb[pl.ds(16, 16)],
                    idxb[pl.ds(K, 16)], idxb[pl.ds(K + 16, 16)])

        def next_idx(i):
            i2 = (i + 2) & (MW - 1)
            return (idxb[pl.ds(i2 * K, 16)], idxb[pl.ds(i2 * K + 16, 16)])

        def xstep(i, carry):
            idx0, idx1, idxn0, idxn1 = carry
            iv = jnp.full((16,), i, jnp.int32)
            coffs = (CXOFF, CXOFF + MW, CXOFF + 2 * MW)
            cvals = [plsc.load_gather(
                arena, [jnp.full((16,), coff + i, jnp.int32)])
                for coff in coffs]
            g0 = [plsc.load_gather(slot(q), [idx0]) - cvals[q]
                  for q in range(3)]
            g1 = [plsc.load_gather(slot(q), [idx1]) - cvals[q]
                  for q in range(3)]
            for q in range(3):
                plsc.store_scatter(ob2.at[q], [sv0, iv], g0[q])
                plsc.store_scatter(ob2.at[q], [sv1, iv], g1[q])
            return (idxn0, idxn1) + next_idx(i)
        lax.fori_loop(0, MW, xstep, idx_init)
        for q in range(3):
            for r in range(4):
                pltpu.sync_copy(ob2.at[q, pl.ds(r * 8, 8), :], gslice(q, r))

        def dgroup(t2, _):
            for par, tbase in ((0, 4), (1, 0)):
                gi = 2 * t2 + par
                ch0 = gi * G
                for q in range(G):
                    twait(ch0 + q, slot(tbase + q), sem_t[q])
                nbase = 4 - tbase

                @pl.when(gi + 1 < C // G)
                def _():
                    for q in range(G):
                        tload(ch0 + G + q, slot(nbase + q), sem_t[q])

                @pl.when(gi >= 1)
                def _():
                    for q in range(G):
                        for r in range(4):
                            pltpu.make_async_copy(
                                ob2.at[q, pl.ds(r * 8, 8), :],
                                gslice(3 + ch0 - G + q, r),
                                sem_o[q]).wait()

                def gbody(i, carry, tbase=tbase):
                    idx0, idx1, idxn0, idxn1 = carry
                    iv = jnp.full((16,), i, jnp.int32)
                    g0 = [plsc.load_gather(slot(tbase + q), [idx0])
                          for q in range(G)]
                    g1 = [plsc.load_gather(slot(tbase + q), [idx1])
                          for q in range(G)]
                    for q in range(G):
                        plsc.store_scatter(ob2.at[q], [sv0, iv], g0[q])
                        plsc.store_scatter(ob2.at[q], [sv1, iv], g1[q])
                    return (idxn0, idxn1) + next_idx(i)
                lax.fori_loop(0, MW, gbody, idx_init)

                for q in range(G):
                    for r in range(4):
                        pltpu.async_copy(ob2.at[q, pl.ds(r * 8, 8), :],
                                         gslice(3 + ch0 + q, r), sem_o[q])
            return 0
        lax.fori_loop(0, C // (2 * G), dgroup, 0)

        for q in range(G):
            for r in range(4):
                pltpu.make_async_copy(ob2.at[q, pl.ds(r * 8, 8), :],
                                      gslice(3 + C - G + q, r),
                                      sem_o[q]).wait()

    return body(xyzc, childc, feats)


def kernel(xyz, child_xyz, feats):
    xyzc = jnp.transpose(xyz, (0, 2, 1)).reshape(-1)
    childc = jnp.transpose(child_xyz, (0, 2, 1)).reshape(-1)
    grouped3, filled1 = _sc_query_group(xyzc, childc, feats.reshape(-1))
    grouped = grouped3.reshape(BS, COUT, K, M).transpose(0, 1, 3, 2)
    filled = filled1.reshape(BS, M, K).astype(jnp.bool_)
    return grouped, filled
